# baseline (speedup 1.0000x reference)
import functools

import jax
import jax.numpy as jnp
from jax import lax
from jax.experimental import pallas as pl
from jax.experimental.pallas import tpu as pltpu
from jax.experimental.pallas import tpu_sc as plsc

B, C, K, H, W = 8, 2, 1024, 512, 512
NUM_CELLS = 2 * B * C
LANES = 16
CHUNK = 128
NCHUNK = K // CHUNK
SUB = CHUNK // LANES
NFIELD = 4

_mesh = plsc.VectorSubcoreMesh(core_axis_name="c", subcore_axis_name="s")


@functools.partial(
    pl.kernel,
    out_type=jax.ShapeDtypeStruct((NUM_CELLS, LANES), jnp.float32),
    mesh=_mesh,
    scratch_types=[
        pltpu.VMEM((NFIELD * NCHUNK, CHUNK), jnp.int32),
        pltpu.VMEM((NFIELD * NCHUNK, CHUNK), jnp.int32),
        pltpu.VMEM((2 * NCHUNK, CHUNK), jnp.int32),
        pltpu.VMEM((2 * NCHUNK, CHUNK), jnp.float32),
        pltpu.VMEM((LANES,), jnp.float32),
        pltpu.SemaphoreType.DMA,
    ],
)
def _bd_loss_sc(pred_hbm, ints0_hbm, ints1_hbm, out_hbm,
                fidx_v, fld_v, pidx_v, vals_v, acc_v, sem):
    cell = lax.axis_index("s") * 2 + lax.axis_index("c")
    comp = lax.div(cell, 16)
    bc = lax.rem(cell, 16)
    base = bc * (H * W)
    good_i = lax.max(1 - comp, lax.rem(cell, 2))

    lane = lax.iota(jnp.int32, LANES)
    i4 = lane * 4
    cell_off = bc * (4 * K)

    for f in range(NFIELD):
        for q in range(NCHUNK):
            r = f * NCHUNK + q
            for t in range(SUB):
                const = (q * CHUNK + t * LANES) * 4 + f
                fidx_v[r, pl.ds(t * LANES, LANES)] = cell_off + const + i4

    nrows = NFIELD * NCHUNK

    @pl.when(comp == 0)
    def _():
        for r in range(nrows):
            pltpu.make_async_copy(
                ints0_hbm.at[fidx_v.at[r]], fld_v.at[r], sem).start()

    @pl.when(comp == 1)
    def _():
        for r in range(nrows):
            pltpu.make_async_copy(
                ints1_hbm.at[fidx_v.at[r]], fld_v.at[r], sem).start()

    for r in range(nrows):
        pltpu.make_async_copy(
            ints0_hbm.at[fidx_v.at[r]], fld_v.at[r], sem).wait()

    vcopies = []
    for side in range(2):
        for q in range(NCHUNK):
            for t in range(SUB):
                s = pl.ds(t * LANES, LANES)
                x = fld_v[(2 * side) * NCHUNK + q, s]
                y = fld_v[(2 * side + 1) * NCHUNK + q, s]
                pidx_v[side * NCHUNK + q, s] = base + x * W + y
            cp = pltpu.make_async_copy(
                pred_hbm.at[pidx_v.at[side * NCHUNK + q]],
                vals_v.at[side * NCHUNK + q], sem)
            cp.start()
            vcopies.append(cp)
    for cp in vcopies:
        cp.wait()

    flip = (jnp.maximum(1 - lane, 0) * good_i).astype(jnp.float32)
    acc = jnp.zeros((LANES,), jnp.float32)
    for q in range(NCHUNK):
        for t in range(SUB):
            s = pl.ds(t * LANES, LANES)
            d = vals_v[q, s] - vals_v[NCHUNK + q, s]
            d2 = d * d
            if q == 0 and t == 0:
                d2 = d2 + flip * (1.0 - 2.0 * d2)
            acc = acc + d2

    acc_v[...] = acc
    pltpu.sync_copy(acc_v, out_hbm.at[cell])


def kernel(prediction, intervals_comp_0, intervals_comp_1):
    partials = _bd_loss_sc(
        prediction.reshape(-1),
        intervals_comp_0.reshape(-1),
        intervals_comp_1.reshape(-1),
    )
    return jnp.sum(partials)


# speedup vs baseline: 3.2768x; 3.2768x over previous
import functools

import jax
import jax.numpy as jnp
from jax import lax
from jax.experimental import pallas as pl
from jax.experimental.pallas import tpu as pltpu
from jax.experimental.pallas import tpu_sc as plsc

B, C, K, H, W = 8, 2, 1024, 512, 512
NUM_CELLS = 2 * B * C
LANES = 16
CHUNK = 128
NCHUNK = K // CHUNK
SUB = CHUNK // LANES

_mesh = plsc.VectorSubcoreMesh(core_axis_name="c", subcore_axis_name="s")


@functools.partial(
    pl.kernel,
    out_type=jax.ShapeDtypeStruct((NUM_CELLS, LANES), jnp.float32),
    mesh=_mesh,
    scratch_types=[
        pltpu.VMEM((K,), jnp.int32),
        pltpu.VMEM((K,), jnp.int32),
        pltpu.VMEM((K,), jnp.int32),
        pltpu.VMEM((K,), jnp.int32),
        pltpu.VMEM((NCHUNK, CHUNK), jnp.int32),
        pltpu.VMEM((NCHUNK, CHUNK), jnp.int32),
        pltpu.VMEM((NCHUNK, CHUNK), jnp.float32),
        pltpu.VMEM((NCHUNK, CHUNK), jnp.float32),
        pltpu.VMEM((LANES,), jnp.float32),
        pltpu.SemaphoreType.DMA,
    ],
)
def _bd_loss_sc(pred_hbm, bx_hbm, by_hbm, dx_hbm, dy_hbm, out_hbm,
                bx_v, by_v, dx_v, dy_v, bidx_v, didx_v,
                bvals_v, dvals_v, acc_v, sem):
    cell = lax.axis_index("s") * 2 + lax.axis_index("c")
    comp = lax.div(cell, 16)
    bc = lax.rem(cell, 16)
    base = bc * (H * W)
    good_i = lax.max(1 - comp, lax.rem(cell, 2))

    pltpu.sync_copy(bx_hbm.at[cell], bx_v)
    pltpu.sync_copy(by_hbm.at[cell], by_v)
    pltpu.sync_copy(dx_hbm.at[cell], dx_v)
    pltpu.sync_copy(dy_hbm.at[cell], dy_v)

    for j in range(NCHUNK):
        for t in range(SUB):
            o = j * CHUNK + t * LANES
            s = pl.ds(t * LANES, LANES)
            bidx_v[j, s] = base + bx_v[pl.ds(o, LANES)] * W + by_v[pl.ds(o, LANES)]
            didx_v[j, s] = base + dx_v[pl.ds(o, LANES)] * W + dy_v[pl.ds(o, LANES)]

    copies = []
    for j in range(NCHUNK):
        copies.append(pltpu.make_async_copy(
            pred_hbm.at[bidx_v.at[j]], bvals_v.at[j], sem))
        copies.append(pltpu.make_async_copy(
            pred_hbm.at[didx_v.at[j]], dvals_v.at[j], sem))
    for cp in copies:
        cp.start()
    for cp in copies:
        cp.wait()

    lane = lax.iota(jnp.int32, LANES)
    flip = (jnp.maximum(1 - lane, 0) * good_i).astype(jnp.float32)
    acc = jnp.zeros((LANES,), jnp.float32)
    for j in range(NCHUNK):
        for t in range(SUB):
            s = pl.ds(t * LANES, LANES)
            d = bvals_v[j, s] - dvals_v[j, s]
            d2 = d * d
            if j == 0 and t == 0:
                d2 = d2 + flip * (1.0 - 2.0 * d2)
            acc = acc + d2

    acc_v[...] = acc
    pltpu.sync_copy(acc_v, out_hbm.at[cell])


def kernel(prediction, intervals_comp_0, intervals_comp_1):
    def field(p, q):
        return jnp.concatenate([
            intervals_comp_0[:, :, :, p, q].reshape(B * C, K),
            intervals_comp_1[:, :, :, p, q].reshape(B * C, K),
        ])

    partials = _bd_loss_sc(
        prediction.reshape(-1),
        field(0, 0), field(0, 1), field(1, 0), field(1, 1),
    )
    return jnp.sum(partials)


# speedup vs baseline: 4.9040x; 1.4966x over previous
import functools

import jax
import jax.numpy as jnp
from jax import lax
from jax.experimental import pallas as pl
from jax.experimental.pallas import tpu as pltpu
from jax.experimental.pallas import tpu_sc as plsc

B, C, K, H, W = 8, 2, 1024, 512, 512
NUM_CELLS = 2 * B * C
LANES = 16
CHUNK = 128
NCHUNK = K // CHUNK
SUB = CHUNK // LANES

_mesh = plsc.VectorSubcoreMesh(core_axis_name="c", subcore_axis_name="s")


def _tiled_idx(x, y):
    return (((x >> 3) << 12) + ((y >> 7) << 10)
            + ((x & 7) << 7) + (y & 127))


@functools.partial(
    pl.kernel,
    out_type=jax.ShapeDtypeStruct((NUM_CELLS, LANES), jnp.float32),
    mesh=_mesh,
    scratch_types=[
        pltpu.VMEM((4, K), jnp.int32),
        pltpu.VMEM((NCHUNK, CHUNK), jnp.int32),
        pltpu.VMEM((NCHUNK, CHUNK), jnp.int32),
        pltpu.VMEM((NCHUNK, CHUNK), jnp.float32),
        pltpu.VMEM((NCHUNK, CHUNK), jnp.float32),
        pltpu.VMEM((LANES,), jnp.float32),
        pltpu.SemaphoreType.DMA,
    ],
)
def _bd_loss_sc(pred_hbm, fld_hbm, out_hbm,
                fld_v, bidx_v, didx_v, bvals_v, dvals_v, acc_v, sem):
    cell = lax.axis_index("s") * 2 + lax.axis_index("c")
    comp = lax.div(cell, 16)
    bc = lax.rem(cell, 16)
    base = bc * (H * W)
    good_i = lax.max(1 - comp, lax.rem(cell, 2))

    for f in range(4):
        pltpu.sync_copy(fld_hbm.at[f * NUM_CELLS + cell], fld_v.at[f])

    for j in range(NCHUNK):
        for t in range(SUB):
            o = pl.ds(j * CHUNK + t * LANES, LANES)
            s = pl.ds(t * LANES, LANES)
            bidx_v[j, s] = base + _tiled_idx(fld_v[0, o], fld_v[1, o])
            didx_v[j, s] = base + _tiled_idx(fld_v[2, o], fld_v[3, o])

    copies = []
    for j in range(NCHUNK):
        copies.append(pltpu.make_async_copy(
            pred_hbm.at[bidx_v.at[j]], bvals_v.at[j], sem))
        copies.append(pltpu.make_async_copy(
            pred_hbm.at[didx_v.at[j]], dvals_v.at[j], sem))
    for cp in copies:
        cp.start()
    for cp in copies:
        cp.wait()

    lane = lax.iota(jnp.int32, LANES)
    flip = (jnp.maximum(1 - lane, 0) * good_i).astype(jnp.float32)
    acc = jnp.zeros((LANES,), jnp.float32)
    for j in range(NCHUNK):
        for t in range(SUB):
            s = pl.ds(t * LANES, LANES)
            d = bvals_v[j, s] - dvals_v[j, s]
            d2 = d * d
            if j == 0 and t == 0:
                d2 = d2 + flip * (1.0 - 2.0 * d2)
            acc = acc + d2

    acc_v[...] = acc
    pltpu.sync_copy(acc_v, out_hbm.at[cell])


def kernel(prediction, intervals_comp_0, intervals_comp_1):
    pred_t = prediction.reshape(B, C, H // 8, 8, W // 128, 128)
    pred_t = pred_t.transpose(0, 1, 2, 4, 3, 5).reshape(-1)

    def field(p, q):
        return jnp.concatenate([
            intervals_comp_0[:, :, :, p, q].reshape(B * C, K),
            intervals_comp_1[:, :, :, p, q].reshape(B * C, K),
        ])

    fld = jnp.concatenate(
        [field(0, 0), field(0, 1), field(1, 0), field(1, 1)])
    partials = _bd_loss_sc(pred_t, fld)
    return jnp.sum(partials)
